# baseline (speedup 1.0000x reference)
import functools

import jax
import jax.numpy as jnp
from jax import lax
from jax.experimental import pallas as pl
from jax.experimental.pallas import tpu as pltpu
from jax.experimental.pallas import tpu_sc as plsc


def _degree_body(vmax, x_ref, deg_ref, acc_ref):
    i = pl.program_id(0)

    @pl.when(i == 0)
    def _init():
        acc_ref[...] = jnp.zeros_like(acc_ref)

    acc_ref[...] += jnp.sum(x_ref[...], axis=0, keepdims=True)

    @pl.when(i == pl.num_programs(0) - 1)
    def _fin():
        deg = jnp.ceil(acc_ref[...]).astype(jnp.int32)
        deg_ref[...] = jnp.clip(deg, 0, vmax)


def _degree(x, vmax, block_rows=200):
    n_rows, n_cols = x.shape
    grid = (pl.cdiv(n_rows, block_rows),)
    out = pl.pallas_call(
        functools.partial(_degree_body, vmax),
        grid=grid,
        in_specs=[pl.BlockSpec((block_rows, n_cols), lambda i: (i, 0))],
        out_specs=pl.BlockSpec((1, n_cols), lambda i: (0, 0)),
        out_shape=jax.ShapeDtypeStruct((1, n_cols), jnp.int32),
        scratch_shapes=[pltpu.VMEM((1, n_cols), jnp.float32)],
    )(x)
    return out.reshape(n_cols)


_LANES = 16
_NBUF = 3


def _gather_add(W, feat, deg, b_per_w=320, chunk=80):
    B, D = feat.shape
    mesh = plsc.VectorSubcoreMesh(core_axis_name="c", subcore_axis_name="s")
    n_chunks = b_per_w // chunk
    vecs_per_row = D // _LANES
    nbuf = min(_NBUF, n_chunks)

    @functools.partial(
        pl.kernel,
        mesh=mesh,
        out_type=jax.ShapeDtypeStruct((B, D), jnp.float32),
        scratch_types=[
            pltpu.VMEM((b_per_w,), jnp.int32),
            [pltpu.VMEM((chunk, D), jnp.float32) for _ in range(nbuf)],
            [pltpu.VMEM((chunk, D), jnp.float32) for _ in range(nbuf)],
            [pltpu.SemaphoreType.DMA for _ in range(nbuf)],
            [pltpu.SemaphoreType.DMA for _ in range(nbuf)],
            [pltpu.SemaphoreType.DMA for _ in range(nbuf)],
        ],
    )
    def k(w_hbm, feat_hbm, deg_hbm, out_hbm,
          idx_v, rows, featb, gsem, fsem, osem):
        wid = lax.axis_index("s") * 2 + lax.axis_index("c")
        base = jnp.minimum(wid * b_per_w, B - b_per_w)

        grs = [None] * nbuf
        frs = [None] * nbuf
        ows = [None] * nbuf

        def start_feat(c):
            buf = c % nbuf
            frs[buf] = pltpu.async_copy(
                feat_hbm.at[pl.ds(base + c * chunk, chunk)], featb[buf], fsem[buf])

        def start_gather(c):
            buf = c % nbuf
            grs[buf] = pltpu.async_copy(
                w_hbm.at[idx_v.at[pl.ds(c * chunk, chunk)]], rows[buf], gsem[buf])

        n_prime = min(nbuf - 1, n_chunks) if nbuf > 1 else 1
        for c in range(n_prime):
            start_feat(c)
        pltpu.sync_copy(deg_hbm.at[pl.ds(base, b_per_w)], idx_v)
        for c in range(n_prime):
            start_gather(c)

        for c in range(n_chunks):
            buf = c % nbuf
            nc = c + nbuf - 1
            if nbuf > 1 and nc < n_chunks:
                nbuf_i = nc % nbuf
                if ows[nbuf_i] is not None:
                    ows[nbuf_i].wait()
                    ows[nbuf_i] = None
                start_gather(nc)
                start_feat(nc)
            grs[buf].wait()
            frs[buf].wait()

            rows_ref = rows[buf]
            featb_ref = featb[buf]

            @plsc.parallel_loop(0, chunk, 1, unroll=8)
            def _row(j):
                for kk in range(vecs_per_row):
                    sl = pl.ds(kk * _LANES, _LANES)
                    plsc.addupdate(featb_ref.at[j, sl], rows_ref[j, sl])

            ows[buf] = pltpu.async_copy(
                featb[buf], out_hbm.at[pl.ds(base + c * chunk, chunk)], osem[buf])
        for d in ows:
            if d is not None:
                d.wait()

    return k(W, feat, deg)


def kernel(x, feat, W):
    deg = _degree(x, W.shape[0] - 1)
    return _gather_add(W, feat, deg)


# speedup vs baseline: 1.0412x; 1.0412x over previous
import functools

import jax
import jax.numpy as jnp
from jax import lax
from jax.experimental import pallas as pl
from jax.experimental.pallas import tpu as pltpu
from jax.experimental.pallas import tpu_sc as plsc


def _degree_body(vmax, x_ref, deg_ref, acc_ref):
    i = pl.program_id(0)

    @pl.when(i == 0)
    def _init():
        acc_ref[...] = jnp.zeros_like(acc_ref)

    acc_ref[...] += jnp.sum(x_ref[...], axis=0, keepdims=True)

    @pl.when(i == pl.num_programs(0) - 1)
    def _fin():
        deg = jnp.ceil(acc_ref[...]).astype(jnp.int32)
        deg_ref[...] = jnp.clip(deg, 0, vmax)


def _degree(x, vmax, block_rows=200):
    n_rows, n_cols = x.shape
    grid = (pl.cdiv(n_rows, block_rows),)
    out = pl.pallas_call(
        functools.partial(_degree_body, vmax),
        grid=grid,
        in_specs=[pl.BlockSpec((block_rows, n_cols), lambda i: (i, 0))],
        out_specs=pl.BlockSpec((1, n_cols), lambda i: (0, 0)),
        out_shape=jax.ShapeDtypeStruct((1, n_cols), jnp.int32),
        scratch_shapes=[pltpu.VMEM((1, n_cols), jnp.float32)],
    )(x)
    return out.reshape(n_cols)


_LANES = 16
_NBUF = 3


def _gather_add(W, feat, deg, b_per_w=320, chunk=80):
    B, D = feat.shape
    mesh = plsc.VectorSubcoreMesh(core_axis_name="c", subcore_axis_name="s")
    n_chunks = b_per_w // chunk
    vecs_per_row = D // _LANES
    nbuf = min(_NBUF, n_chunks)

    @functools.partial(
        pl.kernel,
        mesh=mesh,
        out_type=jax.ShapeDtypeStruct((B, D), jnp.float32),
        scratch_types=[
            pltpu.VMEM((b_per_w,), jnp.int32),
            [pltpu.VMEM((chunk, D), jnp.float32) for _ in range(nbuf)],
            [pltpu.VMEM((chunk, D), jnp.float32) for _ in range(nbuf)],
            [pltpu.SemaphoreType.DMA for _ in range(nbuf)],
            [pltpu.SemaphoreType.DMA for _ in range(nbuf)],
            [pltpu.SemaphoreType.DMA for _ in range(nbuf)],
        ],
    )
    def k(w_hbm, feat_hbm, deg_hbm, out_hbm,
          idx_v, rows, featb, gsem, fsem, osem):
        wid = lax.axis_index("s") * 2 + lax.axis_index("c")
        base = jnp.minimum(wid * b_per_w, B - b_per_w)

        grs = [None] * nbuf
        frs = [None] * nbuf
        ows = [None] * nbuf

        def start_feat(c):
            buf = c % nbuf
            frs[buf] = pltpu.async_copy(
                feat_hbm.at[pl.ds(base + c * chunk, chunk)], featb[buf], fsem[buf])

        def start_gather(c):
            buf = c % nbuf
            grs[buf] = pltpu.async_copy(
                w_hbm.at[idx_v.at[pl.ds(c * chunk, chunk)]], rows[buf], gsem[buf])

        n_prime = min(nbuf - 1, n_chunks) if nbuf > 1 else 1
        for c in range(n_prime):
            start_feat(c)
        pltpu.sync_copy(deg_hbm.at[pl.ds(base, b_per_w)], idx_v)
        for c in range(n_prime):
            start_gather(c)

        for c in range(n_chunks):
            buf = c % nbuf
            nc = c + nbuf - 1
            if nbuf > 1 and nc < n_chunks:
                nbuf_i = nc % nbuf
                if ows[nbuf_i] is not None:
                    ows[nbuf_i].wait()
                    ows[nbuf_i] = None
                start_gather(nc)
                start_feat(nc)
            grs[buf].wait()
            frs[buf].wait()

            rows_ref = rows[buf]
            featb_ref = featb[buf]

            @plsc.parallel_loop(0, chunk, 1, unroll=4)
            def _row(j):
                for kk in range(vecs_per_row):
                    sl = pl.ds(kk * _LANES, _LANES)
                    plsc.addupdate(featb_ref.at[j, sl], rows_ref[j, sl])

            ows[buf] = pltpu.async_copy(
                featb[buf], out_hbm.at[pl.ds(base + c * chunk, chunk)], osem[buf])
        for d in ows:
            if d is not None:
                d.wait()

    return k(W, feat, deg)


def kernel(x, feat, W):
    deg = _degree(x, W.shape[0] - 1)
    return _gather_add(W, feat, deg)
